# baseline (speedup 1.0000x reference)
import functools

import jax
import jax.numpy as jnp
from jax import lax
from jax.experimental import pallas as pl
from jax.experimental.pallas import tpu as pltpu
from jax.experimental.pallas import tpu_sc as plsc

N_BITS = 20
OUT_DIM = 16
BATCH = 16384
NUM_CORES = 2
NUM_SUBCORES = 16
NW = NUM_CORES * NUM_SUBCORES
B_PER_W = BATCH // NW
SUB = 128
N_SUB = B_PER_W // SUB
LANES = 16


def _body(img_hbm, enc_hbm, out_hbm, img_v, idx_v, scale_v, rows_v, sem):
    wid = lax.axis_index("s") * NUM_CORES + lax.axis_index("c")
    base_row = wid * B_PER_W
    pltpu.sync_copy(img_hbm.at[pl.ds(base_row * N_BITS, B_PER_W * N_BITS)], img_v)

    lanes = lax.iota(jnp.int32, LANES)

    def sub_block(s, _):
        sub_base = s * SUB

        def index_chunk(cix, _):
            row0 = sub_base + cix * LANES
            flat0 = (row0 + lanes) * N_BITS
            idx = jnp.zeros((LANES,), jnp.int32)
            acc = jnp.zeros((LANES,), jnp.float32)
            for j in range(N_BITS):
                g = plsc.load_gather(img_v, [flat0 + j])
                bit = jnp.full((LANES,), 1 << j, jnp.int32)
                idx = idx + jnp.where(g > 0, bit, jnp.zeros((LANES,), jnp.int32))
                acc = acc + jnp.abs(g)
            idx_v[pl.ds(cix * LANES, LANES)] = idx
            scale_v[pl.ds(cix * LANES, LANES)] = acc * (1.0 / N_BITS)
            return _

        lax.fori_loop(0, SUB // LANES, index_chunk, 0)

        pltpu.async_copy(enc_hbm.at[idx_v], rows_v, sem).wait()

        def scale_chunk(cix, _):
            rbase = cix * LANES
            svec = scale_v[pl.ds(rbase, LANES)]
            ridx = rbase + lanes
            for c in range(OUT_DIM):
                cvec = jnp.full((LANES,), c, jnp.int32)
                col = plsc.load_gather(rows_v, [ridx, cvec])
                plsc.store_scatter(rows_v, [ridx, cvec], col * svec)
            return _

        lax.fori_loop(0, SUB // LANES, scale_chunk, 0)

        pltpu.sync_copy(rows_v, out_hbm.at[pl.ds(base_row + sub_base, SUB)])
        return _

    lax.fori_loop(0, N_SUB, sub_block, 0)


@jax.jit
def kernel(image, encoding):
    mesh = plsc.VectorSubcoreMesh(
        core_axis_name="c", subcore_axis_name="s",
        num_cores=NUM_CORES, num_subcores=NUM_SUBCORES)
    k = functools.partial(
        pl.kernel,
        out_type=jax.ShapeDtypeStruct((BATCH, OUT_DIM), jnp.float32),
        mesh=mesh,
        scratch_types=[
            pltpu.VMEM((B_PER_W * N_BITS,), jnp.float32),
            pltpu.VMEM((SUB,), jnp.int32),
            pltpu.VMEM((SUB,), jnp.float32),
            pltpu.VMEM((SUB, OUT_DIM), jnp.float32),
            pltpu.SemaphoreType.DMA,
        ],
        compiler_params=pltpu.CompilerParams(
            needs_layout_passes=False, use_tc_tiling_on_sc=False),
    )(_body)
    return k(image.reshape(-1), encoding)
